# baseline (speedup 1.0000x reference)
import functools

import jax
import jax.numpy as jnp
from jax import lax
from jax.experimental import pallas as pl
from jax.experimental.pallas import tpu as pltpu

N = 100000
NPAD = 102400
E = 3200000
DH = 24
BE = 12800
BN = 10240


def _full(shape):
    return pl.BlockSpec(shape, lambda i: tuple(0 for _ in shape))


def _mlp3_pad(h, w1, b1, w2, b2, w3p, b3p):
    f32 = jnp.float32
    h = jnp.maximum(jnp.dot(h, w1, preferred_element_type=f32) + b1, 0.0)
    h = jnp.maximum(jnp.dot(h, w2, preferred_element_type=f32) + b2, 0.0)
    return jnp.dot(h, w3p, preferred_element_type=f32) + b3p


def _ef_body(ea_ref, w1_ref, b1_ref, w2_ref, b2_ref, w3a_ref, b3a_ref,
             w3b_ref, b3b_ref, out_ref):
    f32 = jnp.float32
    h = jnp.maximum(jnp.dot(ea_ref[...], w1_ref[...],
                            preferred_element_type=f32) + b1_ref[...], 0.0)
    h = jnp.maximum(jnp.dot(h, w2_ref[...],
                            preferred_element_type=f32) + b2_ref[...], 0.0)
    out_ref[0, :, :] = jnp.dot(h, w3a_ref[...], preferred_element_type=f32) + b3a_ref[...]
    out_ref[1, :, :] = jnp.dot(h, w3b_ref[...], preferred_element_type=f32) + b3b_ref[...]


def _init_body(x_ref, w1_ref, b1_ref, w2_ref, b2_ref, w3p_ref, b3p_ref, out_ref):
    t = _mlp3_pad(x_ref[...], w1_ref[...], b1_ref[...], w2_ref[...],
                  b2_ref[...], w3p_ref[...], b3p_ref[...])
    out_ref[0, :, :] = t[:, :16]
    out_ref[1, :, :] = t[:, 16:]


def _agg_h(acc_ref):
    s0 = acc_ref[0]
    s1 = acc_ref[1]
    cnt = jnp.maximum(s0[:, 12:13], 1.0)
    return jnp.concatenate([s0[:, :12], s1[:, :12]], axis=1) / cnt


def _upd_body(acc_ref, w1_ref, b1_ref, w2_ref, b2_ref, w3p_ref, b3p_ref, out_ref):
    t = _mlp3_pad(_agg_h(acc_ref), w1_ref[...], b1_ref[...], w2_ref[...],
                  b2_ref[...], w3p_ref[...], b3p_ref[...])
    out_ref[0, :, :] = t[:, :16]
    out_ref[1, :, :] = t[:, 16:]


def _readout_body(acc_ref, w1_ref, b1_ref, w2_ref, b2_ref, out_ref):
    f32 = jnp.float32
    h = _agg_h(acc_ref)
    h = jnp.maximum(jnp.dot(h, w1_ref[...], preferred_element_type=f32) + b1_ref[...], 0.0)
    out_ref[...] = jnp.dot(h, w2_ref[...], preferred_element_type=f32) + b2_ref[...]


def _pad_last(w3, b3):
    d = w3.shape[0]
    w3p = jnp.zeros((d, 32), jnp.float32)
    w3p = w3p.at[:, :12].set(w3[:, :12]).at[:, 16:28].set(w3[:, 12:24])
    b3p = jnp.zeros((32,), jnp.float32)
    b3p = b3p.at[:12].set(b3[:12]).at[16:28].set(b3[12:24]).at[12].set(1.0)
    return w3p, b3p.reshape(1, 32)


def _wargs(ps):
    (w1, b1), (w2, b2), (w3, b3) = ps
    w3p, b3p = _pad_last(w3, b3)
    args = (w1, b1.reshape(1, -1), w2, b2.reshape(1, -1), w3p, b3p)
    specs = [_full(a.shape) for a in args]
    return args, specs


def _ef_call(eap8, ps):
    (w1, b1), (w2, b2), (w3, b3) = ps
    w3p, b3p = _pad_last(w3, b3)
    eye8 = jnp.eye(8, dtype=jnp.float32)
    w1b = jnp.kron(eye8, w1)
    w2b = jnp.kron(eye8, w2)
    w3a = jnp.kron(eye8, w3p[:, :16])
    w3b = jnp.kron(eye8, w3p[:, 16:])
    b1b = jnp.tile(b1, 8).reshape(1, 192)
    b2b = jnp.tile(b2, 8).reshape(1, 192)
    b3a = jnp.tile(b3p[0, :16], 8).reshape(1, 128)
    b3b = jnp.tile(b3p[0, 16:], 8).reshape(1, 128)
    args = (w1b, b1b, w2b, b2b, w3a, b3a, w3b, b3b)
    wspecs = [_full(a.shape) for a in args]
    return pl.pallas_call(
        _ef_body,
        grid=(E // BE,),
        in_specs=[pl.BlockSpec((BE // 8, 24), lambda i: (i, 0))] + wspecs,
        out_specs=pl.BlockSpec((2, BE // 8, 128), lambda i: (0, i, 0)),
        out_shape=jax.ShapeDtypeStruct((2, E // 8, 128), jnp.float32),
    )(eap8, *args)


def _init_call(x, ps):
    args, wspecs = _wargs(ps)
    return pl.pallas_call(
        _init_body,
        grid=(NPAD // BN,),
        in_specs=[pl.BlockSpec((BN, 3), lambda i: (i, 0))] + wspecs,
        out_specs=pl.BlockSpec((2, BN, 16), lambda i: (0, i, 0)),
        out_shape=jax.ShapeDtypeStruct((2, NPAD, 16), jnp.float32),
    )(x, *args)


def _upd_call(acc, ps):
    args, wspecs = _wargs(ps)
    return pl.pallas_call(
        _upd_body,
        grid=(NPAD // BN,),
        in_specs=[pl.BlockSpec((2, BN, 16), lambda i: (0, i, 0))] + wspecs,
        out_specs=pl.BlockSpec((2, BN, 16), lambda i: (0, i, 0)),
        out_shape=jax.ShapeDtypeStruct((2, NPAD, 16), jnp.float32),
    )(acc, *args)


def _readout_call(acc, ps):
    (w1, b1), (w2, b2) = ps
    args = (w1, b1.reshape(1, -1), w2, b2.reshape(1, -1))
    wspecs = [_full(a.shape) for a in args]
    return pl.pallas_call(
        _readout_body,
        grid=(NPAD // BN,),
        in_specs=[pl.BlockSpec((2, BN, 16), lambda i: (0, i, 0))] + wspecs,
        out_specs=pl.BlockSpec((BN, 3), lambda i: (i, 0)),
        out_shape=jax.ShapeDtypeStruct((NPAD, 3), jnp.float32),
    )(acc, *args)


from jax.experimental.pallas import tpu_sc as plsc

K = 512
RPC = K // 128
NSUB = 16
CPS = 400
CPS_LAST = 250
RPS = NPAD // NSUB
ZB = 256


def _sc_body(tbl, ei3, efr, out, src_v, dst_v, g_v, ef_v, acc_sh,
             sem_src, sem_dst, sem_e, sem_g0, sem_g1):
    c = lax.axis_index("c")
    s = lax.axis_index("s")
    f32 = jnp.float32

    def zrow(i, _):
        g_v[0, i, :] = jnp.zeros((16,), f32)
        return 0
    lax.fori_loop(0, ZB, zrow, 0)

    def zcp(i, _):
        pltpu.sync_copy(g_v.at[0, pl.ds(0, ZB)],
                        acc_sh.at[pl.ds(s * RPS + i * ZB, ZB)])
        return 0
    lax.fori_loop(0, RPS // ZB, zcp, 0)
    plsc.subcore_barrier()

    nchunks = jnp.where(s == NSUB - 1, CPS_LAST, CPS)
    ebase = s * CPS * K
    sems = (sem_g0, sem_g1)

    pltpu.sync_copy(ei3.at[0, pl.ds(ebase, K)], src_v.at[0])
    pltpu.sync_copy(ei3.at[1, pl.ds(ebase, K)], dst_v.at[0])
    pltpu.async_copy(tbl.at[c].at[src_v.at[0]], g_v.at[0], sem_g0)
    pltpu.async_copy(efr.at[c, pl.ds(ebase // 8, K // 8)], ef_v, sem_e)

    def part(ci, p):
        have_next = ci + 1 < nchunks

        @pl.when(have_next)
        def _prefetch_idx():
            r = ebase + (ci + 1) * K
            pltpu.async_copy(ei3.at[0, pl.ds(r, K)], src_v.at[1 - p], sem_src)
            pltpu.async_copy(ei3.at[1, pl.ds(r, K)], dst_v.at[1 - p], sem_dst)

        pltpu.make_async_copy(tbl.at[c].at[src_v.at[p]], g_v.at[p], sems[p]).wait()
        pltpu.make_async_copy(efr.at[c, pl.ds(ebase // 8, K // 8)], ef_v,
                              sem_e).wait()

        def mul(r, _):
            for q in range(8):
                e = 8 * r + q
                g_v[p, e, :] = g_v[p, e, :] * ef_v[r, pl.ds(16 * q, 16)]
            return 0
        lax.fori_loop(0, K // 8, mul, 0, unroll=2)

        @pl.when(have_next)
        def _next_gather():
            pltpu.make_async_copy(ei3.at[0, pl.ds(0, K)], src_v.at[1 - p],
                                  sem_src).wait()
            pltpu.async_copy(tbl.at[c].at[src_v.at[1 - p]], g_v.at[1 - p],
                             sems[1 - p])
            pltpu.async_copy(efr.at[c, pl.ds((ebase + (ci + 1) * K) // 8, K // 8)],
                             ef_v, sem_e)
            pltpu.make_async_copy(ei3.at[1, pl.ds(0, K)], dst_v.at[1 - p],
                                  sem_dst).wait()

        pltpu.sync_copy(g_v.at[p, pl.ds(0, K)], acc_sh.at[dst_v.at[p]], add=True)
        return ci

    def two(i, _):
        part(2 * i, 0)
        part(2 * i + 1, 1)
        return 0
    lax.fori_loop(0, nchunks // 2, two, 0)
    plsc.subcore_barrier()

    pltpu.sync_copy(acc_sh.at[pl.ds(s * RPS, RPS)], out.at[c, pl.ds(s * RPS, RPS)])


def _sc_call(tbl, ei3, efr):
    mesh = plsc.VectorSubcoreMesh(core_axis_name="c", subcore_axis_name="s")
    f = pl.kernel(
        _sc_body,
        out_type=jax.ShapeDtypeStruct((2, NPAD, 16), jnp.float32),
        mesh=mesh,
        compiler_params=pltpu.CompilerParams(use_tc_tiling_on_sc=False),
        scratch_types=[
            pltpu.VMEM((2, K), jnp.int32),
            pltpu.VMEM((2, K), jnp.int32),
            pltpu.VMEM((2, K, 16), jnp.float32),
            pltpu.VMEM((K // 8, 128), jnp.float32),
            pltpu.VMEM_SHARED((NPAD, 16), jnp.float32),
            pltpu.SemaphoreType.DMA,
            pltpu.SemaphoreType.DMA,
            pltpu.SemaphoreType.DMA,
            pltpu.SemaphoreType.DMA,
            pltpu.SemaphoreType.DMA,
        ],
    )
    return f(tbl, ei3, efr)


def kernel(x, edge_index, edge_attr, params):
    ei3 = edge_index
    xp = jnp.concatenate([x, jnp.zeros((NPAD - N, 3), jnp.float32)])

    layers = params["layers"]
    tbl = _init_call(xp, layers[0]["density"])
    acc = None
    ea8 = edge_attr.reshape(E // 8, 24)
    for s in range(3):
        ef = _ef_call(ea8, layers[s]["edge"])
        acc = _sc_call(tbl, ei3, ef)
        if s < 2:
            tbl = _upd_call(acc, layers[s + 1]["density"])
    return _readout_call(acc, params["readout"])[:N]


# speedup vs baseline: 1.0006x; 1.0006x over previous
import functools

import jax
import jax.numpy as jnp
from jax import lax
from jax.experimental import pallas as pl
from jax.experimental.pallas import tpu as pltpu

N = 100000
NPAD = 102400
E = 3200000
DH = 24
BE = 12800
BN = 10240


def _full(shape):
    return pl.BlockSpec(shape, lambda i: tuple(0 for _ in shape))


def _mlp3_pad(h, w1, b1, w2, b2, w3p, b3p):
    f32 = jnp.float32
    h = jnp.maximum(jnp.dot(h, w1, preferred_element_type=f32) + b1, 0.0)
    h = jnp.maximum(jnp.dot(h, w2, preferred_element_type=f32) + b2, 0.0)
    return jnp.dot(h, w3p, preferred_element_type=f32) + b3p


def _ef_body(ea_ref, w1_ref, b1_ref, w2_ref, b2_ref, w3a_ref, b3a_ref,
             w3b_ref, b3b_ref, out_ref):
    f32 = jnp.float32
    h = jnp.maximum(jnp.dot(ea_ref[...], w1_ref[...],
                            preferred_element_type=f32) + b1_ref[...], 0.0)
    h = jnp.maximum(jnp.dot(h, w2_ref[...],
                            preferred_element_type=f32) + b2_ref[...], 0.0)
    out_ref[0, :, :] = jnp.dot(h, w3a_ref[...], preferred_element_type=f32) + b3a_ref[...]
    out_ref[1, :, :] = jnp.dot(h, w3b_ref[...], preferred_element_type=f32) + b3b_ref[...]


def _init_body(x_ref, w1_ref, b1_ref, w2_ref, b2_ref, w3p_ref, b3p_ref, out_ref):
    t = _mlp3_pad(x_ref[...], w1_ref[...], b1_ref[...], w2_ref[...],
                  b2_ref[...], w3p_ref[...], b3p_ref[...])
    out_ref[0, :, :] = t[:, :16]
    out_ref[1, :, :] = t[:, 16:]


def _agg_h(acc_ref):
    s0 = acc_ref[0]
    s1 = acc_ref[1]
    cnt = jnp.maximum(s0[:, 12:13], 1.0)
    return jnp.concatenate([s0[:, :12], s1[:, :12]], axis=1) / cnt


def _upd_body(acc_ref, w1_ref, b1_ref, w2_ref, b2_ref, w3p_ref, b3p_ref, out_ref):
    t = _mlp3_pad(_agg_h(acc_ref), w1_ref[...], b1_ref[...], w2_ref[...],
                  b2_ref[...], w3p_ref[...], b3p_ref[...])
    out_ref[0, :, :] = t[:, :16]
    out_ref[1, :, :] = t[:, 16:]


def _readout_body(acc_ref, w1_ref, b1_ref, w2_ref, b2_ref, out_ref):
    f32 = jnp.float32
    h = _agg_h(acc_ref)
    h = jnp.maximum(jnp.dot(h, w1_ref[...], preferred_element_type=f32) + b1_ref[...], 0.0)
    out_ref[...] = jnp.dot(h, w2_ref[...], preferred_element_type=f32) + b2_ref[...]


def _pad_last(w3, b3):
    d = w3.shape[0]
    w3p = jnp.zeros((d, 32), jnp.float32)
    w3p = w3p.at[:, :12].set(w3[:, :12]).at[:, 16:28].set(w3[:, 12:24])
    b3p = jnp.zeros((32,), jnp.float32)
    b3p = b3p.at[:12].set(b3[:12]).at[16:28].set(b3[12:24]).at[12].set(1.0)
    return w3p, b3p.reshape(1, 32)


def _wargs(ps):
    (w1, b1), (w2, b2), (w3, b3) = ps
    w3p, b3p = _pad_last(w3, b3)
    args = (w1, b1.reshape(1, -1), w2, b2.reshape(1, -1), w3p, b3p)
    specs = [_full(a.shape) for a in args]
    return args, specs


def _ef_call(eap8, ps):
    (w1, b1), (w2, b2), (w3, b3) = ps
    w3p, b3p = _pad_last(w3, b3)
    eye8 = jnp.eye(8, dtype=jnp.float32)
    w1b = jnp.kron(eye8, w1)
    w2b = jnp.kron(eye8, w2)
    w3a = jnp.kron(eye8, w3p[:, :16])
    w3b = jnp.kron(eye8, w3p[:, 16:])
    b1b = jnp.tile(b1, 8).reshape(1, 192)
    b2b = jnp.tile(b2, 8).reshape(1, 192)
    b3a = jnp.tile(b3p[0, :16], 8).reshape(1, 128)
    b3b = jnp.tile(b3p[0, 16:], 8).reshape(1, 128)
    args = (w1b, b1b, w2b, b2b, w3a, b3a, w3b, b3b)
    wspecs = [_full(a.shape) for a in args]
    return pl.pallas_call(
        _ef_body,
        grid=(E // BE,),
        in_specs=[pl.BlockSpec((BE // 8, 24), lambda i: (i, 0))] + wspecs,
        out_specs=pl.BlockSpec((2, BE // 8, 128), lambda i: (0, i, 0)),
        out_shape=jax.ShapeDtypeStruct((2, E // 8, 128), jnp.float32),
    )(eap8, *args)


def _init_call(x, ps):
    args, wspecs = _wargs(ps)
    return pl.pallas_call(
        _init_body,
        grid=(NPAD // BN,),
        in_specs=[pl.BlockSpec((BN, 3), lambda i: (i, 0))] + wspecs,
        out_specs=pl.BlockSpec((2, BN, 16), lambda i: (0, i, 0)),
        out_shape=jax.ShapeDtypeStruct((2, NPAD, 16), jnp.float32),
    )(x, *args)


def _upd_call(acc, ps):
    args, wspecs = _wargs(ps)
    return pl.pallas_call(
        _upd_body,
        grid=(NPAD // BN,),
        in_specs=[pl.BlockSpec((2, BN, 16), lambda i: (0, i, 0))] + wspecs,
        out_specs=pl.BlockSpec((2, BN, 16), lambda i: (0, i, 0)),
        out_shape=jax.ShapeDtypeStruct((2, NPAD, 16), jnp.float32),
    )(acc, *args)


def _readout_call(acc, ps):
    (w1, b1), (w2, b2) = ps
    args = (w1, b1.reshape(1, -1), w2, b2.reshape(1, -1))
    wspecs = [_full(a.shape) for a in args]
    return pl.pallas_call(
        _readout_body,
        grid=(NPAD // BN,),
        in_specs=[pl.BlockSpec((2, BN, 16), lambda i: (0, i, 0))] + wspecs,
        out_specs=pl.BlockSpec((BN, 3), lambda i: (i, 0)),
        out_shape=jax.ShapeDtypeStruct((NPAD, 3), jnp.float32),
    )(acc, *args)


from jax.experimental.pallas import tpu_sc as plsc

K = 512
RPC = K // 128
NSUB = 16
CPS = 400
CPS_LAST = 250
RPS = NPAD // NSUB
ZB = 256


def _sc_body(tbl, srcl, dstl, efr, out, src_v, dst_v, g_v, ef_v, acc_sh,
             sem_src, sem_dst, sem_e, sem_g0, sem_g1):
    c = lax.axis_index("c")
    s = lax.axis_index("s")
    f32 = jnp.float32

    def zrow(i, _):
        g_v[0, i, :] = jnp.zeros((16,), f32)
        return 0
    lax.fori_loop(0, ZB, zrow, 0)

    def zcp(i, _):
        pltpu.sync_copy(g_v.at[0, pl.ds(0, ZB)],
                        acc_sh.at[pl.ds(s * RPS + i * ZB, ZB)])
        return 0
    lax.fori_loop(0, RPS // ZB, zcp, 0)
    plsc.subcore_barrier()

    nchunks = jnp.where(s == NSUB - 1, CPS_LAST, CPS)
    ebase = s * CPS * K
    sems = (sem_g0, sem_g1)

    pltpu.sync_copy(srcl.at[pl.ds(ebase, K)], src_v.at[0])
    pltpu.sync_copy(dstl.at[pl.ds(ebase, K)], dst_v.at[0])
    pltpu.async_copy(tbl.at[c].at[src_v.at[0]], g_v.at[0], sem_g0)
    pltpu.async_copy(efr.at[c, pl.ds(ebase // 8, K // 8)], ef_v, sem_e)

    def part(ci, p):
        have_next = ci + 1 < nchunks

        @pl.when(have_next)
        def _prefetch_idx():
            r = ebase + (ci + 1) * K
            pltpu.async_copy(srcl.at[pl.ds(r, K)], src_v.at[1 - p], sem_src)
            pltpu.async_copy(dstl.at[pl.ds(r, K)], dst_v.at[1 - p], sem_dst)

        pltpu.make_async_copy(tbl.at[c].at[src_v.at[p]], g_v.at[p], sems[p]).wait()
        pltpu.make_async_copy(efr.at[c, pl.ds(ebase // 8, K // 8)], ef_v,
                              sem_e).wait()

        def mul(r, _):
            for q in range(8):
                e = 8 * r + q
                g_v[p, e, :] = g_v[p, e, :] * ef_v[r, pl.ds(16 * q, 16)]
            return 0
        lax.fori_loop(0, K // 8, mul, 0, unroll=2)

        @pl.when(have_next)
        def _next_gather():
            pltpu.make_async_copy(srcl.at[pl.ds(0, K)], src_v.at[1 - p],
                                  sem_src).wait()
            pltpu.async_copy(tbl.at[c].at[src_v.at[1 - p]], g_v.at[1 - p],
                             sems[1 - p])
            pltpu.async_copy(efr.at[c, pl.ds((ebase + (ci + 1) * K) // 8, K // 8)],
                             ef_v, sem_e)
            pltpu.make_async_copy(dstl.at[pl.ds(0, K)], dst_v.at[1 - p],
                                  sem_dst).wait()

        pltpu.sync_copy(g_v.at[p, pl.ds(0, K)], acc_sh.at[dst_v.at[p]], add=True)
        return ci

    def two(i, _):
        part(2 * i, 0)
        part(2 * i + 1, 1)
        return 0
    lax.fori_loop(0, nchunks // 2, two, 0)
    plsc.subcore_barrier()

    pltpu.sync_copy(acc_sh.at[pl.ds(s * RPS, RPS)], out.at[c, pl.ds(s * RPS, RPS)])


def _sc_call(tbl, srcl, dstl, efr):
    mesh = plsc.VectorSubcoreMesh(core_axis_name="c", subcore_axis_name="s")
    f = pl.kernel(
        _sc_body,
        out_type=jax.ShapeDtypeStruct((2, NPAD, 16), jnp.float32),
        mesh=mesh,
        compiler_params=pltpu.CompilerParams(use_tc_tiling_on_sc=False),
        scratch_types=[
            pltpu.VMEM((2, K), jnp.int32),
            pltpu.VMEM((2, K), jnp.int32),
            pltpu.VMEM((2, K, 16), jnp.float32),
            pltpu.VMEM((K // 8, 128), jnp.float32),
            pltpu.VMEM_SHARED((NPAD, 16), jnp.float32),
            pltpu.SemaphoreType.DMA,
            pltpu.SemaphoreType.DMA,
            pltpu.SemaphoreType.DMA,
            pltpu.SemaphoreType.DMA,
            pltpu.SemaphoreType.DMA,
        ],
    )
    return f(tbl, srcl, dstl, efr)


def kernel(x, edge_index, edge_attr, params):
    srcl = edge_index[0]
    dstl = edge_index[1]
    xp = jnp.concatenate([x, jnp.zeros((NPAD - N, 3), jnp.float32)])

    layers = params["layers"]
    tbl = _init_call(xp, layers[0]["density"])
    acc = None
    ea8 = edge_attr.reshape(E // 8, 24)
    for s in range(3):
        ef = _ef_call(ea8, layers[s]["edge"])
        acc = _sc_call(tbl, srcl, dstl, ef)
        if s < 2:
            tbl = _upd_call(acc, layers[s + 1]["density"])
    return _readout_call(acc, params["readout"])[:N]


# speedup vs baseline: 1.9996x; 1.9984x over previous
import functools

import jax
import jax.numpy as jnp
from jax import lax
from jax.experimental import pallas as pl
from jax.experimental.pallas import tpu as pltpu

N = 100000
NPAD = 102400
E = 3200000
DH = 24
BE = 12800
BN = 10240


def _full(shape):
    return pl.BlockSpec(shape, lambda i: tuple(0 for _ in shape))


def _mlp3_pad(h, w1, b1, w2, b2, w3p, b3p):
    f32 = jnp.float32
    h = jnp.maximum(jnp.dot(h, w1, preferred_element_type=f32) + b1, 0.0)
    h = jnp.maximum(jnp.dot(h, w2, preferred_element_type=f32) + b2, 0.0)
    return jnp.dot(h, w3p, preferred_element_type=f32) + b3p


def _ef_body(ea_ref, w1_ref, b1_ref, w2_ref, b2_ref, w3a_ref, b3a_ref,
             w3b_ref, b3b_ref, out_ref):
    f32 = jnp.float32
    h = jnp.maximum(jnp.dot(ea_ref[...], w1_ref[...],
                            preferred_element_type=f32) + b1_ref[...], 0.0)
    h = jnp.maximum(jnp.dot(h, w2_ref[...],
                            preferred_element_type=f32) + b2_ref[...], 0.0)
    out_ref[0, :, :] = jnp.dot(h, w3a_ref[...], preferred_element_type=f32) + b3a_ref[...]
    out_ref[1, :, :] = jnp.dot(h, w3b_ref[...], preferred_element_type=f32) + b3b_ref[...]


def _init_body(x_ref, w1_ref, b1_ref, w2_ref, b2_ref, w3p_ref, b3p_ref, out_ref):
    t = _mlp3_pad(x_ref[...], w1_ref[...], b1_ref[...], w2_ref[...],
                  b2_ref[...], w3p_ref[...], b3p_ref[...])
    out_ref[0, :, :] = t[:, :16]
    out_ref[1, :, :] = t[:, 16:]


def _agg_h(acc_ref):
    s0 = acc_ref[0]
    s1 = acc_ref[1]
    cnt = jnp.maximum(s0[:, 12:13], 1.0)
    return jnp.concatenate([s0[:, :12], s1[:, :12]], axis=1) / cnt


def _upd_body(acc_ref, w1_ref, b1_ref, w2_ref, b2_ref, w3p_ref, b3p_ref, out_ref):
    t = _mlp3_pad(_agg_h(acc_ref), w1_ref[...], b1_ref[...], w2_ref[...],
                  b2_ref[...], w3p_ref[...], b3p_ref[...])
    out_ref[0, :, :] = t[:, :16]
    out_ref[1, :, :] = t[:, 16:]


def _readout_body(acc_ref, w1_ref, b1_ref, w2_ref, b2_ref, out_ref):
    f32 = jnp.float32
    h = _agg_h(acc_ref)
    h = jnp.maximum(jnp.dot(h, w1_ref[...], preferred_element_type=f32) + b1_ref[...], 0.0)
    out_ref[...] = jnp.dot(h, w2_ref[...], preferred_element_type=f32) + b2_ref[...]


def _pad_last(w3, b3):
    d = w3.shape[0]
    w3p = jnp.zeros((d, 32), jnp.float32)
    w3p = w3p.at[:, :12].set(w3[:, :12]).at[:, 16:28].set(w3[:, 12:24])
    b3p = jnp.zeros((32,), jnp.float32)
    b3p = b3p.at[:12].set(b3[:12]).at[16:28].set(b3[12:24]).at[12].set(1.0)
    return w3p, b3p.reshape(1, 32)


def _wargs(ps):
    (w1, b1), (w2, b2), (w3, b3) = ps
    w3p, b3p = _pad_last(w3, b3)
    args = (w1, b1.reshape(1, -1), w2, b2.reshape(1, -1), w3p, b3p)
    specs = [_full(a.shape) for a in args]
    return args, specs


def _ef_call(eap8, ps):
    (w1, b1), (w2, b2), (w3, b3) = ps
    w3p, b3p = _pad_last(w3, b3)
    eye8 = jnp.eye(8, dtype=jnp.float32)
    w1b = jnp.kron(eye8, w1)
    w2b = jnp.kron(eye8, w2)
    w3a = jnp.kron(eye8, w3p[:, :16])
    w3b = jnp.kron(eye8, w3p[:, 16:])
    b1b = jnp.tile(b1, 8).reshape(1, 192)
    b2b = jnp.tile(b2, 8).reshape(1, 192)
    b3a = jnp.tile(b3p[0, :16], 8).reshape(1, 128)
    b3b = jnp.tile(b3p[0, 16:], 8).reshape(1, 128)
    args = (w1b, b1b, w2b, b2b, w3a, b3a, w3b, b3b)
    wspecs = [_full(a.shape) for a in args]
    return pl.pallas_call(
        _ef_body,
        grid=(E // BE,),
        in_specs=[pl.BlockSpec((BE // 8, 24), lambda i: (i, 0))] + wspecs,
        out_specs=pl.BlockSpec((2, BE // 8, 128), lambda i: (0, i, 0)),
        out_shape=jax.ShapeDtypeStruct((2, E // 8, 128), jnp.float32),
    )(eap8, *args)


def _init_call(x, ps):
    args, wspecs = _wargs(ps)
    return pl.pallas_call(
        _init_body,
        grid=(NPAD // BN,),
        in_specs=[pl.BlockSpec((BN, 3), lambda i: (i, 0))] + wspecs,
        out_specs=pl.BlockSpec((2, BN, 16), lambda i: (0, i, 0)),
        out_shape=jax.ShapeDtypeStruct((2, NPAD, 16), jnp.float32),
    )(x, *args)


def _upd_call(acc, ps):
    args, wspecs = _wargs(ps)
    return pl.pallas_call(
        _upd_body,
        grid=(NPAD // BN,),
        in_specs=[pl.BlockSpec((2, BN, 16), lambda i: (0, i, 0))] + wspecs,
        out_specs=pl.BlockSpec((2, BN, 16), lambda i: (0, i, 0)),
        out_shape=jax.ShapeDtypeStruct((2, NPAD, 16), jnp.float32),
    )(acc, *args)


def _readout_call(acc, ps):
    (w1, b1), (w2, b2) = ps
    args = (w1, b1.reshape(1, -1), w2, b2.reshape(1, -1))
    wspecs = [_full(a.shape) for a in args]
    return pl.pallas_call(
        _readout_body,
        grid=(NPAD // BN,),
        in_specs=[pl.BlockSpec((2, BN, 16), lambda i: (0, i, 0))] + wspecs,
        out_specs=pl.BlockSpec((BN, 3), lambda i: (i, 0)),
        out_shape=jax.ShapeDtypeStruct((NPAD, 3), jnp.float32),
    )(acc, *args)


from jax.experimental.pallas import tpu_sc as plsc

K = 512
RPC = K // 128
NSUB = 16
CPS = 400
CPS_LAST = 250
RPS = NPAD // NSUB
ZB = 256


def _sc_body(tbl, srcl, dstl, efr, out, src_v, dst_v, g_v, ef_v, acc_sh,
             sem_src, sem_dst, sem_e, sem_g0, sem_g1):
    c = lax.axis_index("c")
    s = lax.axis_index("s")
    f32 = jnp.float32

    def zrow(i, _):
        g_v[0, i, :] = jnp.zeros((16,), f32)
        return 0
    lax.fori_loop(0, ZB, zrow, 0)

    def zcp(i, _):
        pltpu.sync_copy(g_v.at[0, pl.ds(0, ZB)],
                        acc_sh.at[pl.ds(s * RPS + i * ZB, ZB)])
        return 0
    lax.fori_loop(0, RPS // ZB, zcp, 0)
    plsc.subcore_barrier()

    nchunks = jnp.where(s == NSUB - 1, CPS_LAST, CPS)
    ebase = s * CPS * K
    sems = (sem_g0, sem_g1)

    pltpu.sync_copy(srcl.at[pl.ds(ebase, K)], src_v.at[0])
    pltpu.sync_copy(dstl.at[pl.ds(ebase, K)], dst_v.at[0])
    pltpu.async_copy(tbl.at[c].at[src_v.at[0]], g_v.at[0], sem_g0)
    pltpu.async_copy(efr.at[c, pl.ds(ebase // 8, K // 8)], ef_v, sem_e)

    def part(ci, p):
        have_next = ci + 1 < nchunks

        @pl.when(have_next)
        def _prefetch_idx():
            r = ebase + (ci + 1) * K
            pltpu.async_copy(srcl.at[pl.ds(r, K)], src_v.at[1 - p], sem_src)
            pltpu.async_copy(dstl.at[pl.ds(r, K)], dst_v.at[1 - p], sem_dst)

        pltpu.make_async_copy(tbl.at[c].at[src_v.at[p]], g_v.at[p], sems[p]).wait()
        pltpu.make_async_copy(efr.at[c, pl.ds(ebase // 8, K // 8)], ef_v,
                              sem_e).wait()

        def mul(r, _):
            for q in range(8):
                e = 8 * r + q
                g_v[p, e, :] = g_v[p, e, :] * ef_v[r, pl.ds(16 * q, 16)]
            return 0
        lax.fori_loop(0, K // 8, mul, 0, unroll=2)

        @pl.when(have_next)
        def _next_gather():
            pltpu.make_async_copy(srcl.at[pl.ds(0, K)], src_v.at[1 - p],
                                  sem_src).wait()
            pltpu.async_copy(tbl.at[c].at[src_v.at[1 - p]], g_v.at[1 - p],
                             sems[1 - p])
            pltpu.async_copy(efr.at[c, pl.ds((ebase + (ci + 1) * K) // 8, K // 8)],
                             ef_v, sem_e)
            pltpu.make_async_copy(dstl.at[pl.ds(0, K)], dst_v.at[1 - p],
                                  sem_dst).wait()

        pltpu.sync_copy(g_v.at[p, pl.ds(0, K)], acc_sh.at[dst_v.at[p]], add=True)
        return ci

    def two(i, _):
        part(2 * i, 0)
        part(2 * i + 1, 1)
        return 0
    lax.fori_loop(0, nchunks // 2, two, 0)
    plsc.subcore_barrier()

    pltpu.sync_copy(acc_sh.at[pl.ds(s * RPS, RPS)], out.at[c, pl.ds(s * RPS, RPS)])


def _sc_call(tbl, srcl, dstl, efr):
    mesh = plsc.VectorSubcoreMesh(core_axis_name="c", subcore_axis_name="s")
    f = pl.kernel(
        _sc_body,
        out_type=jax.ShapeDtypeStruct((2, NPAD, 16), jnp.float32),
        mesh=mesh,
        compiler_params=pltpu.CompilerParams(use_tc_tiling_on_sc=False),
        scratch_types=[
            pltpu.VMEM((2, K), jnp.int32),
            pltpu.VMEM((2, K), jnp.int32),
            pltpu.VMEM((2, K, 16), jnp.float32),
            pltpu.VMEM((K // 8, 128), jnp.float32),
            pltpu.VMEM_SHARED((NPAD, 16), jnp.float32),
            pltpu.SemaphoreType.DMA,
            pltpu.SemaphoreType.DMA,
            pltpu.SemaphoreType.DMA,
            pltpu.SemaphoreType.DMA,
            pltpu.SemaphoreType.DMA,
        ],
    )
    return f(tbl, srcl, dstl, efr)


def kernel(x, edge_index, edge_attr, params):
    srcl = edge_index[0]
    dstl = edge_index[1]
    xp = jnp.concatenate([x, jnp.zeros((NPAD - N, 3), jnp.float32)])

    layers = params["layers"]
    tbl = _init_call(xp, layers[0]["density"])
    acc = None
    ea8 = jnp.concatenate([edge_attr[q::8, :] for q in range(8)], axis=1)
    for s in range(3):
        ef = _ef_call(ea8, layers[s]["edge"])
        acc = _sc_call(tbl, srcl, dstl, ef)
        if s < 2:
            tbl = _upd_call(acc, layers[s + 1]["density"])
    return _readout_call(acc, params["readout"])[:N]


# speedup vs baseline: 2.3054x; 1.1529x over previous
import functools

import jax
import jax.numpy as jnp
from jax import lax
from jax.experimental import pallas as pl
from jax.experimental.pallas import tpu as pltpu

N = 100000
NPAD = 102400
E = 3200000
DH = 24
BE = 12800
BN = 10240


def _full(shape):
    return pl.BlockSpec(shape, lambda i: tuple(0 for _ in shape))


def _mlp3_pad(h, w1, b1, w2, b2, w3p, b3p):
    f32 = jnp.float32
    h = jnp.maximum(jnp.dot(h, w1, preferred_element_type=f32) + b1, 0.0)
    h = jnp.maximum(jnp.dot(h, w2, preferred_element_type=f32) + b2, 0.0)
    return jnp.dot(h, w3p, preferred_element_type=f32) + b3p


def _ef_body(ea_ref, w1_ref, b1_ref, w2_ref, b2_ref, w3a_ref, b3a_ref,
             w3b_ref, b3b_ref, out_ref):
    f32 = jnp.float32
    h = jnp.maximum(jnp.dot(ea_ref[...], w1_ref[...],
                            preferred_element_type=f32) + b1_ref[...], 0.0)
    h = jnp.maximum(jnp.dot(h, w2_ref[...],
                            preferred_element_type=f32) + b2_ref[...], 0.0)
    out_ref[0, :, :] = jnp.dot(h, w3a_ref[...], preferred_element_type=f32) + b3a_ref[...]
    out_ref[1, :, :] = jnp.dot(h, w3b_ref[...], preferred_element_type=f32) + b3b_ref[...]


def _init_body(x_ref, w1_ref, b1_ref, w2_ref, b2_ref, w3p_ref, b3p_ref, out_ref):
    t = _mlp3_pad(x_ref[...], w1_ref[...], b1_ref[...], w2_ref[...],
                  b2_ref[...], w3p_ref[...], b3p_ref[...])
    out_ref[0, :, :] = t[:, :16]
    out_ref[1, :, :] = t[:, 16:]


def _agg_h(acc_ref):
    s0 = acc_ref[0]
    s1 = acc_ref[1]
    cnt = jnp.maximum(s0[:, 12:13], 1.0)
    return jnp.concatenate([s0[:, :12], s1[:, :12]], axis=1) / cnt


def _upd_body(acc_ref, w1_ref, b1_ref, w2_ref, b2_ref, w3p_ref, b3p_ref, out_ref):
    t = _mlp3_pad(_agg_h(acc_ref), w1_ref[...], b1_ref[...], w2_ref[...],
                  b2_ref[...], w3p_ref[...], b3p_ref[...])
    out_ref[0, :, :] = t[:, :16]
    out_ref[1, :, :] = t[:, 16:]


def _readout_body(acc_ref, w1_ref, b1_ref, w2_ref, b2_ref, out_ref):
    f32 = jnp.float32
    h = _agg_h(acc_ref)
    h = jnp.maximum(jnp.dot(h, w1_ref[...], preferred_element_type=f32) + b1_ref[...], 0.0)
    out_ref[...] = jnp.dot(h, w2_ref[...], preferred_element_type=f32) + b2_ref[...]


def _pad_last(w3, b3):
    d = w3.shape[0]
    w3p = jnp.zeros((d, 32), jnp.float32)
    w3p = w3p.at[:, :12].set(w3[:, :12]).at[:, 16:28].set(w3[:, 12:24])
    b3p = jnp.zeros((32,), jnp.float32)
    b3p = b3p.at[:12].set(b3[:12]).at[16:28].set(b3[12:24]).at[12].set(1.0)
    return w3p, b3p.reshape(1, 32)


def _wargs(ps):
    (w1, b1), (w2, b2), (w3, b3) = ps
    w3p, b3p = _pad_last(w3, b3)
    args = (w1, b1.reshape(1, -1), w2, b2.reshape(1, -1), w3p, b3p)
    specs = [_full(a.shape) for a in args]
    return args, specs


def _ef_call(eap8, ps):
    (w1, b1), (w2, b2), (w3, b3) = ps
    w3p, b3p = _pad_last(w3, b3)
    eye8 = jnp.eye(8, dtype=jnp.float32)
    w1b = (w1[:, None, None, :] * eye8[None, :, :, None]).reshape(24, 192)
    w2b = jnp.kron(eye8, w2)
    w3a = jnp.kron(eye8, w3p[:, :16])
    w3b = jnp.kron(eye8, w3p[:, 16:])
    b1b = jnp.tile(b1, 8).reshape(1, 192)
    b2b = jnp.tile(b2, 8).reshape(1, 192)
    b3a = jnp.tile(b3p[0, :16], 8).reshape(1, 128)
    b3b = jnp.tile(b3p[0, 16:], 8).reshape(1, 128)
    args = (w1b, b1b, w2b, b2b, w3a, b3a, w3b, b3b)
    wspecs = [_full(a.shape) for a in args]
    return pl.pallas_call(
        _ef_body,
        grid=(E // BE,),
        in_specs=[pl.BlockSpec((BE // 8, 24), lambda i: (i, 0))] + wspecs,
        out_specs=pl.BlockSpec((2, BE // 8, 128), lambda i: (0, i, 0)),
        out_shape=jax.ShapeDtypeStruct((2, E // 8, 128), jnp.float32),
    )(eap8, *args)


def _init_call(x, ps):
    args, wspecs = _wargs(ps)
    return pl.pallas_call(
        _init_body,
        grid=(NPAD // BN,),
        in_specs=[pl.BlockSpec((BN, 3), lambda i: (i, 0))] + wspecs,
        out_specs=pl.BlockSpec((2, BN, 16), lambda i: (0, i, 0)),
        out_shape=jax.ShapeDtypeStruct((2, NPAD, 16), jnp.float32),
    )(x, *args)


def _upd_call(acc, ps):
    args, wspecs = _wargs(ps)
    return pl.pallas_call(
        _upd_body,
        grid=(NPAD // BN,),
        in_specs=[pl.BlockSpec((2, BN, 16), lambda i: (0, i, 0))] + wspecs,
        out_specs=pl.BlockSpec((2, BN, 16), lambda i: (0, i, 0)),
        out_shape=jax.ShapeDtypeStruct((2, NPAD, 16), jnp.float32),
    )(acc, *args)


def _readout_call(acc, ps):
    (w1, b1), (w2, b2) = ps
    args = (w1, b1.reshape(1, -1), w2, b2.reshape(1, -1))
    wspecs = [_full(a.shape) for a in args]
    return pl.pallas_call(
        _readout_body,
        grid=(NPAD // BN,),
        in_specs=[pl.BlockSpec((2, BN, 16), lambda i: (0, i, 0))] + wspecs,
        out_specs=pl.BlockSpec((BN, 3), lambda i: (i, 0)),
        out_shape=jax.ShapeDtypeStruct((NPAD, 3), jnp.float32),
    )(acc, *args)


from jax.experimental.pallas import tpu_sc as plsc

K = 512
RPC = K // 128
NSUB = 16
CPS = 400
CPS_LAST = 250
RPS = NPAD // NSUB
ZB = 256


def _sc_body(tbl, srcl, dstl, efr, out, src_v, dst_v, g_v, ef_v, acc_sh,
             sem_src, sem_dst, sem_e, sem_g0, sem_g1):
    c = lax.axis_index("c")
    s = lax.axis_index("s")
    f32 = jnp.float32

    def zrow(i, _):
        g_v[0, i, :] = jnp.zeros((16,), f32)
        return 0
    lax.fori_loop(0, ZB, zrow, 0)

    def zcp(i, _):
        pltpu.sync_copy(g_v.at[0, pl.ds(0, ZB)],
                        acc_sh.at[pl.ds(s * RPS + i * ZB, ZB)])
        return 0
    lax.fori_loop(0, RPS // ZB, zcp, 0)
    plsc.subcore_barrier()

    nchunks = jnp.where(s == NSUB - 1, CPS_LAST, CPS)
    ebase = s * CPS * K
    sems = (sem_g0, sem_g1)

    pltpu.sync_copy(srcl.at[pl.ds(ebase, K)], src_v.at[0])
    pltpu.sync_copy(dstl.at[pl.ds(ebase, K)], dst_v.at[0])
    pltpu.async_copy(tbl.at[c].at[src_v.at[0]], g_v.at[0], sem_g0)
    pltpu.async_copy(efr.at[c, pl.ds(ebase // 8, K // 8)], ef_v, sem_e)

    def part(ci, p):
        have_next = ci + 1 < nchunks

        @pl.when(have_next)
        def _prefetch_idx():
            r = ebase + (ci + 1) * K
            pltpu.async_copy(srcl.at[pl.ds(r, K)], src_v.at[1 - p], sem_src)
            pltpu.async_copy(dstl.at[pl.ds(r, K)], dst_v.at[1 - p], sem_dst)

        pltpu.make_async_copy(tbl.at[c].at[src_v.at[p]], g_v.at[p], sems[p]).wait()
        pltpu.make_async_copy(efr.at[c, pl.ds(ebase // 8, K // 8)], ef_v,
                              sem_e).wait()

        def mul(r, _):
            for q in range(8):
                e = 8 * r + q
                g_v[p, e, :] = g_v[p, e, :] * ef_v[r, pl.ds(16 * q, 16)]
            return 0
        lax.fori_loop(0, K // 8, mul, 0, unroll=2)

        @pl.when(have_next)
        def _next_gather():
            pltpu.make_async_copy(srcl.at[pl.ds(0, K)], src_v.at[1 - p],
                                  sem_src).wait()
            pltpu.async_copy(tbl.at[c].at[src_v.at[1 - p]], g_v.at[1 - p],
                             sems[1 - p])
            pltpu.async_copy(efr.at[c, pl.ds((ebase + (ci + 1) * K) // 8, K // 8)],
                             ef_v, sem_e)
            pltpu.make_async_copy(dstl.at[pl.ds(0, K)], dst_v.at[1 - p],
                                  sem_dst).wait()

        pltpu.sync_copy(g_v.at[p, pl.ds(0, K)], acc_sh.at[dst_v.at[p]], add=True)
        return ci

    def two(i, _):
        part(2 * i, 0)
        part(2 * i + 1, 1)
        return 0
    lax.fori_loop(0, nchunks // 2, two, 0)
    plsc.subcore_barrier()

    pltpu.sync_copy(acc_sh.at[pl.ds(s * RPS, RPS)], out.at[c, pl.ds(s * RPS, RPS)])


def _sc_call(tbl, srcl, dstl, efr):
    mesh = plsc.VectorSubcoreMesh(core_axis_name="c", subcore_axis_name="s")
    f = pl.kernel(
        _sc_body,
        out_type=jax.ShapeDtypeStruct((2, NPAD, 16), jnp.float32),
        mesh=mesh,
        compiler_params=pltpu.CompilerParams(use_tc_tiling_on_sc=False),
        scratch_types=[
            pltpu.VMEM((2, K), jnp.int32),
            pltpu.VMEM((2, K), jnp.int32),
            pltpu.VMEM((2, K, 16), jnp.float32),
            pltpu.VMEM((K // 8, 128), jnp.float32),
            pltpu.VMEM_SHARED((NPAD, 16), jnp.float32),
            pltpu.SemaphoreType.DMA,
            pltpu.SemaphoreType.DMA,
            pltpu.SemaphoreType.DMA,
            pltpu.SemaphoreType.DMA,
            pltpu.SemaphoreType.DMA,
        ],
    )
    return f(tbl, srcl, dstl, efr)


def kernel(x, edge_index, edge_attr, params):
    srcl = edge_index[0]
    dstl = edge_index[1]
    xp = jnp.concatenate([x, jnp.zeros((NPAD - N, 3), jnp.float32)])

    layers = params["layers"]
    tbl = _init_call(xp, layers[0]["density"])
    acc = None
    ea8 = edge_attr.T.reshape(3, E // 8, 8).transpose(1, 0, 2).reshape(E // 8, 24)
    for s in range(3):
        ef = _ef_call(ea8, layers[s]["edge"])
        acc = _sc_call(tbl, srcl, dstl, ef)
        if s < 2:
            tbl = _upd_call(acc, layers[s + 1]["density"])
    return _readout_call(acc, params["readout"])[:N]


# speedup vs baseline: 3.2752x; 1.4207x over previous
import functools

import jax
import jax.numpy as jnp
from jax import lax
from jax.experimental import pallas as pl
from jax.experimental.pallas import tpu as pltpu

N = 100000
NPAD = 102400
E = 3200000
DH = 24
BE = 12800
BN = 10240


def _full(shape):
    return pl.BlockSpec(shape, lambda i: tuple(0 for _ in shape))


def _mlp3_pad(h, w1, b1, w2, b2, w3p, b3p):
    f32 = jnp.float32
    h = jnp.maximum(jnp.dot(h, w1, preferred_element_type=f32) + b1, 0.0)
    h = jnp.maximum(jnp.dot(h, w2, preferred_element_type=f32) + b2, 0.0)
    return jnp.dot(h, w3p, preferred_element_type=f32) + b3p


def _ef_body(ea_ref, w1_ref, b1_ref, w2_ref, b2_ref, w3a_ref, b3a_ref,
             w3b_ref, b3b_ref, out_ref):
    f32 = jnp.float32
    h = jnp.maximum(jnp.dot(ea_ref[...], w1_ref[...],
                            preferred_element_type=f32) + b1_ref[...], 0.0)
    h = jnp.maximum(jnp.dot(h, w2_ref[...],
                            preferred_element_type=f32) + b2_ref[...], 0.0)
    out_ref[0, :, :] = jnp.dot(h, w3a_ref[...], preferred_element_type=f32) + b3a_ref[...]
    out_ref[1, :, :] = jnp.dot(h, w3b_ref[...], preferred_element_type=f32) + b3b_ref[...]


def _init_body(x_ref, w1_ref, b1_ref, w2_ref, b2_ref, w3p_ref, b3p_ref, out_ref):
    t = _mlp3_pad(x_ref[...], w1_ref[...], b1_ref[...], w2_ref[...],
                  b2_ref[...], w3p_ref[...], b3p_ref[...])
    out_ref[0, :, :] = t[:, :16]
    out_ref[1, :, :] = t[:, 16:]


def _agg_h(acc_ref):
    s0 = acc_ref[0]
    s1 = acc_ref[1]
    cnt = jnp.maximum(s0[:, 12:13], 1.0)
    return jnp.concatenate([s0[:, :12], s1[:, :12]], axis=1) / cnt


def _upd_body(acc_ref, w1_ref, b1_ref, w2_ref, b2_ref, w3p_ref, b3p_ref, out_ref):
    t = _mlp3_pad(_agg_h(acc_ref), w1_ref[...], b1_ref[...], w2_ref[...],
                  b2_ref[...], w3p_ref[...], b3p_ref[...])
    out_ref[0, :, :] = t[:, :16]
    out_ref[1, :, :] = t[:, 16:]


def _readout_body(acc_ref, w1_ref, b1_ref, w2_ref, b2_ref, out_ref):
    f32 = jnp.float32
    h = _agg_h(acc_ref)
    h = jnp.maximum(jnp.dot(h, w1_ref[...], preferred_element_type=f32) + b1_ref[...], 0.0)
    out_ref[...] = jnp.dot(h, w2_ref[...], preferred_element_type=f32) + b2_ref[...]


def _pad_last(w3, b3):
    d = w3.shape[0]
    w3p = jnp.zeros((d, 32), jnp.float32)
    w3p = w3p.at[:, :12].set(w3[:, :12]).at[:, 16:28].set(w3[:, 12:24])
    b3p = jnp.zeros((32,), jnp.float32)
    b3p = b3p.at[:12].set(b3[:12]).at[16:28].set(b3[12:24]).at[12].set(1.0)
    return w3p, b3p.reshape(1, 32)


def _wargs(ps):
    (w1, b1), (w2, b2), (w3, b3) = ps
    w3p, b3p = _pad_last(w3, b3)
    args = (w1, b1.reshape(1, -1), w2, b2.reshape(1, -1), w3p, b3p)
    specs = [_full(a.shape) for a in args]
    return args, specs


def _ef_call(eap8, ps):
    (w1, b1), (w2, b2), (w3, b3) = ps
    w3p, b3p = _pad_last(w3, b3)
    eye8 = jnp.eye(8, dtype=jnp.float32)
    w1b = (w1[:, None, None, :] * eye8[None, :, :, None]).reshape(24, 192)
    w2b = jnp.kron(eye8, w2)
    w3a = jnp.kron(eye8, w3p[:, :16])
    w3b = jnp.kron(eye8, w3p[:, 16:])
    b1b = jnp.tile(b1, 8).reshape(1, 192)
    b2b = jnp.tile(b2, 8).reshape(1, 192)
    b3a = jnp.tile(b3p[0, :16], 8).reshape(1, 128)
    b3b = jnp.tile(b3p[0, 16:], 8).reshape(1, 128)
    args = (w1b, b1b, w2b, b2b, w3a, b3a, w3b, b3b)
    wspecs = [_full(a.shape) for a in args]
    return pl.pallas_call(
        _ef_body,
        grid=(E // BE,),
        in_specs=[pl.BlockSpec((BE // 8, 24), lambda i: (i, 0))] + wspecs,
        out_specs=pl.BlockSpec((2, BE // 8, 128), lambda i: (0, i, 0)),
        out_shape=jax.ShapeDtypeStruct((2, E // 8, 128), jnp.float32),
    )(eap8, *args)


def _init_call(x, ps):
    args, wspecs = _wargs(ps)
    return pl.pallas_call(
        _init_body,
        grid=(NPAD // BN,),
        in_specs=[pl.BlockSpec((BN, 3), lambda i: (i, 0))] + wspecs,
        out_specs=pl.BlockSpec((2, BN, 16), lambda i: (0, i, 0)),
        out_shape=jax.ShapeDtypeStruct((2, NPAD, 16), jnp.float32),
    )(x, *args)


def _upd_call(acc, ps):
    args, wspecs = _wargs(ps)
    return pl.pallas_call(
        _upd_body,
        grid=(NPAD // BN,),
        in_specs=[pl.BlockSpec((2, BN, 16), lambda i: (0, i, 0))] + wspecs,
        out_specs=pl.BlockSpec((2, BN, 16), lambda i: (0, i, 0)),
        out_shape=jax.ShapeDtypeStruct((2, NPAD, 16), jnp.float32),
    )(acc, *args)


def _readout_call(acc, ps):
    (w1, b1), (w2, b2) = ps
    args = (w1, b1.reshape(1, -1), w2, b2.reshape(1, -1))
    wspecs = [_full(a.shape) for a in args]
    return pl.pallas_call(
        _readout_body,
        grid=(NPAD // BN,),
        in_specs=[pl.BlockSpec((2, BN, 16), lambda i: (0, i, 0))] + wspecs,
        out_specs=pl.BlockSpec((BN, 3), lambda i: (i, 0)),
        out_shape=jax.ShapeDtypeStruct((NPAD, 3), jnp.float32),
    )(acc, *args)


from jax.experimental.pallas import tpu_sc as plsc

K = 512
RPC = K // 128
NSUB = 16
CPS = 400
CPS_LAST = 250
RPS = NPAD // NSUB
ZB = 256


def _sc_body(tbl, srcl, dstl, efr, out, src_v, dst_v, g_v, ef_v, acc_sh,
             sem_src, sem_dst, sem_e, sem_g0, sem_g1):
    c = lax.axis_index("c")
    s = lax.axis_index("s")
    f32 = jnp.float32

    def zrow(i, _):
        g_v[0, i, :] = jnp.zeros((16,), f32)
        return 0
    lax.fori_loop(0, ZB, zrow, 0)

    def zcp(i, _):
        pltpu.sync_copy(g_v.at[0, pl.ds(0, ZB)],
                        acc_sh.at[pl.ds(s * RPS + i * ZB, ZB)])
        return 0
    lax.fori_loop(0, RPS // ZB, zcp, 0)
    plsc.subcore_barrier()

    nchunks = jnp.where(s == NSUB - 1, CPS_LAST, CPS)
    ebase = s * CPS * K
    sems = (sem_g0, sem_g1)

    pltpu.sync_copy(srcl.at[pl.ds(ebase, K)], src_v.at[0])
    pltpu.sync_copy(dstl.at[pl.ds(ebase, K)], dst_v.at[0])
    pltpu.async_copy(tbl.at[c].at[src_v.at[0]], g_v.at[0], sem_g0)
    pltpu.async_copy(efr.at[c, pl.ds(ebase // 8, K // 8)], ef_v, sem_e)

    def part(ci, p):
        have_next = ci + 1 < nchunks

        @pl.when(have_next)
        def _prefetch_idx():
            r = ebase + (ci + 1) * K
            pltpu.async_copy(srcl.at[pl.ds(r, K)], src_v.at[1 - p], sem_src)
            pltpu.async_copy(dstl.at[pl.ds(r, K)], dst_v.at[1 - p], sem_dst)

        pltpu.make_async_copy(tbl.at[c].at[src_v.at[p]], g_v.at[p], sems[p]).wait()
        pltpu.make_async_copy(efr.at[c, pl.ds(ebase // 8, K // 8)], ef_v,
                              sem_e).wait()

        @plsc.parallel_loop(0, K // 8, unroll=4)
        def mul(r):
            for q in range(8):
                e = 8 * r + q
                g_v[p, e, :] = g_v[p, e, :] * ef_v[r, pl.ds(16 * q, 16)]

        @pl.when(have_next)
        def _next_gather():
            pltpu.make_async_copy(srcl.at[pl.ds(0, K)], src_v.at[1 - p],
                                  sem_src).wait()
            pltpu.async_copy(tbl.at[c].at[src_v.at[1 - p]], g_v.at[1 - p],
                             sems[1 - p])
            pltpu.async_copy(efr.at[c, pl.ds((ebase + (ci + 1) * K) // 8, K // 8)],
                             ef_v, sem_e)
            pltpu.make_async_copy(dstl.at[pl.ds(0, K)], dst_v.at[1 - p],
                                  sem_dst).wait()

        pltpu.sync_copy(g_v.at[p, pl.ds(0, K)], acc_sh.at[dst_v.at[p]], add=True)
        return ci

    def two(i, _):
        part(2 * i, 0)
        part(2 * i + 1, 1)
        return 0
    lax.fori_loop(0, nchunks // 2, two, 0)
    plsc.subcore_barrier()

    pltpu.sync_copy(acc_sh.at[pl.ds(s * RPS, RPS)], out.at[c, pl.ds(s * RPS, RPS)])


def _sc_call(tbl, srcl, dstl, efr):
    mesh = plsc.VectorSubcoreMesh(core_axis_name="c", subcore_axis_name="s")
    f = pl.kernel(
        _sc_body,
        out_type=jax.ShapeDtypeStruct((2, NPAD, 16), jnp.float32),
        mesh=mesh,
        compiler_params=pltpu.CompilerParams(use_tc_tiling_on_sc=False),
        scratch_types=[
            pltpu.VMEM((2, K), jnp.int32),
            pltpu.VMEM((2, K), jnp.int32),
            pltpu.VMEM((2, K, 16), jnp.float32),
            pltpu.VMEM((K // 8, 128), jnp.float32),
            pltpu.VMEM_SHARED((NPAD, 16), jnp.float32),
            pltpu.SemaphoreType.DMA,
            pltpu.SemaphoreType.DMA,
            pltpu.SemaphoreType.DMA,
            pltpu.SemaphoreType.DMA,
            pltpu.SemaphoreType.DMA,
        ],
    )
    return f(tbl, srcl, dstl, efr)


def kernel(x, edge_index, edge_attr, params):
    srcl = edge_index[0]
    dstl = edge_index[1]
    xp = jnp.concatenate([x, jnp.zeros((NPAD - N, 3), jnp.float32)])

    layers = params["layers"]
    tbl = _init_call(xp, layers[0]["density"])
    acc = None
    ea8 = edge_attr.T.reshape(3, E // 8, 8).transpose(1, 0, 2).reshape(E // 8, 24)
    for s in range(3):
        ef = _ef_call(ea8, layers[s]["edge"])
        acc = _sc_call(tbl, srcl, dstl, ef)
        if s < 2:
            tbl = _upd_call(acc, layers[s + 1]["density"])
    return _readout_call(acc, params["readout"])[:N]
